# baseline (speedup 1.0000x reference)
import functools

import jax
import jax.numpy as jnp
from jax import lax
from jax.experimental import pallas as pl
from jax.experimental.pallas import tpu as pltpu
from jax.experimental.pallas import tpu_sc as plsc

_T = 1000
_L = 16


def _body(b_per_w, num_cores, t_hbm, alpha_hbm, sigma_hbm, out_a_hbm,
          out_s_hbm, idx_v, alpha_v, sigma_v, oa_v, os_v, sem_in, sem_out):
    wid = lax.axis_index("s") * num_cores + lax.axis_index("c")
    base = wid * b_per_w
    c1 = pltpu.async_copy(t_hbm.at[pl.ds(base, b_per_w)], idx_v, sem_in)
    c2 = pltpu.async_copy(alpha_hbm, alpha_v, sem_in)
    c3 = pltpu.async_copy(sigma_hbm, sigma_v, sem_in)
    c1.wait()
    c2.wait()
    c3.wait()
    @plsc.parallel_loop(0, b_per_w // _L, unroll=4)
    def step(j):
        o = j * _L
        t = idx_v[pl.ds(o, _L)]
        g = jnp.maximum(jnp.minimum(t - 1, _T - 1), 0)
        oa_v[pl.ds(o, _L)] = plsc.load_gather(alpha_v, [g])
        os_v[pl.ds(o, _L)] = plsc.load_gather(sigma_v, [g])
    c4 = pltpu.async_copy(oa_v, out_a_hbm.at[pl.ds(base, b_per_w)], sem_out)
    c5 = pltpu.async_copy(os_v, out_s_hbm.at[pl.ds(base, b_per_w)], sem_out)
    c4.wait()
    c5.wait()


def kernel(t_idx, alpha, sigma):
    batch = t_idx.shape[0]
    info = plsc.get_sparse_core_info()
    num_cores = 1
    num_subcores = 4
    nw = num_cores * num_subcores
    b_per_w = batch // nw
    mesh = plsc.VectorSubcoreMesh(core_axis_name="c", subcore_axis_name="s",
                                  num_cores=num_cores,
                                  num_subcores=num_subcores)
    f32 = jnp.float32
    run = pl.kernel(
        functools.partial(_body, b_per_w, num_cores),
        out_type=(jax.ShapeDtypeStruct((batch,), f32),
                  jax.ShapeDtypeStruct((batch,), f32)),
        mesh=mesh,
        scratch_types=[
            pltpu.VMEM((b_per_w,), jnp.int32),
            pltpu.VMEM((_T,), f32),
            pltpu.VMEM((_T,), f32),
            pltpu.VMEM((b_per_w,), f32),
            pltpu.VMEM((b_per_w,), f32),
            pltpu.SemaphoreType.DMA,
            pltpu.SemaphoreType.DMA,
        ],
        compiler_params=pltpu.CompilerParams(needs_layout_passes=False),
    )
    return run(t_idx, alpha, sigma)


# speedup vs baseline: 1.0163x; 1.0163x over previous
import functools

import jax
import jax.numpy as jnp
from jax import lax
from jax.experimental import pallas as pl
from jax.experimental.pallas import tpu as pltpu
from jax.experimental.pallas import tpu_sc as plsc

_T = 1000
_L = 16
_NUM_CORES = 1
_NUM_SUBCORES = 8


def _body(b_per_w, t_hbm, alpha_hbm, sigma_hbm, out_a_hbm, out_s_hbm,
          idx_v, alpha_v, sigma_v, oa_v, os_v, sem_in, sem_out):
    wid = lax.axis_index("s") * _NUM_CORES + lax.axis_index("c")
    base = wid * b_per_w
    c1 = pltpu.async_copy(t_hbm.at[pl.ds(base, b_per_w)], idx_v, sem_in)
    c2 = pltpu.async_copy(alpha_hbm, alpha_v, sem_in)
    c3 = pltpu.async_copy(sigma_hbm, sigma_v, sem_in)
    c1.wait()
    c2.wait()
    c3.wait()

    @plsc.parallel_loop(0, b_per_w // _L, unroll=4)
    def step(j):
        o = j * _L
        t = idx_v[pl.ds(o, _L)]
        g = jnp.maximum(jnp.minimum(t - 1, _T - 1), 0)
        oa_v[pl.ds(o, _L)] = plsc.load_gather(alpha_v, [g])
        os_v[pl.ds(o, _L)] = plsc.load_gather(sigma_v, [g])

    c4 = pltpu.async_copy(oa_v, out_a_hbm.at[pl.ds(base, b_per_w)], sem_out)
    c5 = pltpu.async_copy(os_v, out_s_hbm.at[pl.ds(base, b_per_w)], sem_out)
    c4.wait()
    c5.wait()


def kernel(t_idx, alpha, sigma):
    batch = t_idx.shape[0]
    b_per_w = batch // (_NUM_CORES * _NUM_SUBCORES)
    mesh = plsc.VectorSubcoreMesh(core_axis_name="c", subcore_axis_name="s",
                                  num_cores=_NUM_CORES,
                                  num_subcores=_NUM_SUBCORES)
    f32 = jnp.float32
    run = pl.kernel(
        functools.partial(_body, b_per_w),
        out_type=(jax.ShapeDtypeStruct((batch,), f32),
                  jax.ShapeDtypeStruct((batch,), f32)),
        mesh=mesh,
        scratch_types=[
            pltpu.VMEM((b_per_w,), jnp.int32),
            pltpu.VMEM((_T,), f32),
            pltpu.VMEM((_T,), f32),
            pltpu.VMEM((b_per_w,), f32),
            pltpu.VMEM((b_per_w,), f32),
            pltpu.SemaphoreType.DMA,
            pltpu.SemaphoreType.DMA,
        ],
        compiler_params=pltpu.CompilerParams(needs_layout_passes=False),
    )
    return run(t_idx, alpha, sigma)


# speedup vs baseline: 1.0167x; 1.0004x over previous
import functools

import jax
import jax.numpy as jnp
from jax import lax
from jax.experimental import pallas as pl
from jax.experimental.pallas import tpu as pltpu
from jax.experimental.pallas import tpu_sc as plsc

_T = 1000
_L = 16
_NUM_CORES = 1
_NUM_SUBCORES = 8


def _body(b_per_w, t_hbm, alpha_hbm, sigma_hbm, out_a_hbm, out_s_hbm,
          idx_v, alpha_v, sigma_v, oa_v, os_v, sem_in, sem_out):
    wid = lax.axis_index("s") * _NUM_CORES + lax.axis_index("c")
    base = wid * b_per_w
    c1 = pltpu.async_copy(t_hbm.at[pl.ds(base, b_per_w)], idx_v, sem_in)
    c2 = pltpu.async_copy(alpha_hbm, alpha_v, sem_in)
    c3 = pltpu.async_copy(sigma_hbm, sigma_v, sem_in)
    c1.wait()
    c2.wait()
    c3.wait()

    @plsc.parallel_loop(0, b_per_w // _L, unroll=4)
    def step(j):
        o = j * _L
        t = idx_v[pl.ds(o, _L)]
        g = jnp.maximum(jnp.minimum(t - 1, _T - 1), 0)
        oa_v[pl.ds(o, _L)] = plsc.load_gather(alpha_v, [g])
        os_v[pl.ds(o, _L)] = plsc.load_gather(sigma_v, [g])

    c4 = pltpu.async_copy(oa_v, out_a_hbm.at[pl.ds(base, b_per_w)], sem_out)
    c5 = pltpu.async_copy(os_v, out_s_hbm.at[pl.ds(base, b_per_w)], sem_out)
    c4.wait()
    c5.wait()


def kernel(t_idx, alpha, sigma):
    batch = t_idx.shape[0]
    b_per_w = batch // (_NUM_CORES * _NUM_SUBCORES)
    mesh = plsc.VectorSubcoreMesh(core_axis_name="c", subcore_axis_name="s",
                                  num_cores=_NUM_CORES,
                                  num_subcores=_NUM_SUBCORES)
    f32 = jnp.float32
    run = pl.kernel(
        functools.partial(_body, b_per_w),
        out_type=(jax.ShapeDtypeStruct((batch,), f32),
                  jax.ShapeDtypeStruct((batch,), f32)),
        mesh=mesh,
        scratch_types=[
            pltpu.VMEM((b_per_w,), jnp.int32),
            pltpu.VMEM((_T,), f32),
            pltpu.VMEM((_T,), f32),
            pltpu.VMEM((b_per_w,), f32),
            pltpu.VMEM((b_per_w,), f32),
            pltpu.SemaphoreType.DMA,
            pltpu.SemaphoreType.DMA,
        ],
        compiler_params=pltpu.CompilerParams(needs_layout_passes=False, disable_bounds_checks=True, disable_semaphore_checks=True),
    )
    return run(t_idx, alpha, sigma)
